# baseline (speedup 1.0000x reference)
import functools

import jax
import jax.numpy as jnp
from jax import lax
from jax.experimental import pallas as pl
from jax.experimental.pallas import tpu as pltpu
from jax.experimental.pallas import tpu_sc as plsc

_ROWS = 64
_NROWS = 2048
_N = 32768
_NW = 32
_RPW = _NROWS // _NW


def _argmax_zero_block(x_ref, o_ref, idx_ref):
    b = x_ref[...]
    n = b.shape[1]
    m = jnp.max(b, axis=1, keepdims=True)
    iota = jax.lax.broadcasted_iota(jnp.int32, b.shape, 1)
    idx = jnp.min(jnp.where(b == m, iota, jnp.int32(n)), axis=1, keepdims=True)
    row0 = pl.program_id(0) * b.shape[0]
    row = row0 + jax.lax.broadcasted_iota(jnp.int32, (b.shape[0], 1), 0)
    idx_ref[...] = row * n + idx
    o_ref[...] = jnp.zeros(o_ref.shape, jnp.int32)


_sc_mesh = plsc.VectorSubcoreMesh(core_axis_name="c", subcore_axis_name="s")


@functools.partial(
    pl.kernel,
    mesh=_sc_mesh,
    out_type=(),
    scratch_types=[
        pltpu.VMEM((_RPW,), jnp.int32),
        pltpu.VMEM((_RPW,), jnp.int32),
        pltpu.SemaphoreType.DMA,
    ],
)
def _scatter_ones(out_hbm, fidx_hbm, idx_v, ones_v, sem):
    wid = lax.axis_index("s") * 2 + lax.axis_index("c")
    base = wid * _RPW
    pltpu.sync_copy(fidx_hbm.at[pl.ds(base, _RPW)], idx_v)
    for i in range(_RPW // 16):
        ones_v[pl.ds(i * 16, 16)] = jnp.ones((16,), jnp.int32)
    pltpu.async_copy(ones_v, out_hbm.at[idx_v], sem).wait()


def kernel(x):
    B, R, N = x.shape
    rows = B * R
    xf = x.reshape(rows, N)
    zeros, fidx = pl.pallas_call(
        _argmax_zero_block,
        grid=(rows // _ROWS,),
        in_specs=[pl.BlockSpec((_ROWS, N), lambda i: (i, 0))],
        out_specs=[
            pl.BlockSpec((_ROWS, N), lambda i: (i, 0)),
            pl.BlockSpec((_ROWS, 1), lambda i: (i, 0)),
        ],
        out_shape=[
            jax.ShapeDtypeStruct((rows, N), jnp.int32),
            jax.ShapeDtypeStruct((rows, 1), jnp.int32),
        ],
    )(xf)
    zref = jax.new_ref(zeros.reshape(rows * N))
    _scatter_ones(zref, fidx.reshape(rows))
    return zref[...].reshape(B, R, N)


# speedup vs baseline: 3.8565x; 3.8565x over previous
import jax
import jax.numpy as jnp
from jax.experimental import pallas as pl

_ROWS = 64


def _hardmax_block(x_ref, o_ref):
    b = x_ref[...]
    n = b.shape[1]
    m = jnp.max(b, axis=1, keepdims=True)
    iota = jax.lax.broadcasted_iota(jnp.int32, b.shape, 1)
    idx = jnp.min(jnp.where(b == m, iota, jnp.int32(n)), axis=1)
    o_ref[...] = jnp.zeros(o_ref.shape, jnp.int32)
    lane = jax.lax.broadcasted_iota(jnp.int32, (1, 128), 1)
    for r in range(o_ref.shape[0]):
        base = (idx[r] // 128) * 128
        chunk = (lane == (idx[r] - base)).astype(jnp.int32)
        o_ref[pl.ds(r, 1), pl.ds(pl.multiple_of(base, 128), 128)] = chunk


def kernel(x):
    B, R, N = x.shape
    rows = B * R
    xf = x.reshape(rows, N)
    out = pl.pallas_call(
        _hardmax_block,
        grid=(rows // _ROWS,),
        in_specs=[pl.BlockSpec((_ROWS, N), lambda i: (i, 0))],
        out_specs=pl.BlockSpec((_ROWS, N), lambda i: (i, 0)),
        out_shape=jax.ShapeDtypeStruct((rows, N), jnp.int32),
    )(xf)
    return out.reshape(B, R, N)
